# Initial kernel scaffold; baseline (speedup 1.0000x reference)
import jax
import jax.numpy as jnp
from jax.experimental import pallas as pl


def kernel(x, edge_index, Ws0, Wn0, b0, Ws1, Wn1, b1, Ws2, Wn2, b2):
    raise NotImplementedError("write your pallas kernel here")


# speedup vs baseline: 4.7279x; 4.7279x over previous
import jax
import jax.numpy as jnp
from jax import lax
from jax.experimental import pallas as pl
from jax.experimental.pallas import tpu as pltpu
from jax.experimental.pallas import tpu_sc as plsc

N_NODES = 10000
N_PAD = 10240
N_EDGES = 320000
NUM_TILES = 16
NUM_CORES = 2
EDGE_BLK = 128
NBLK = 160
E_PAD = NUM_TILES * NBLK * EDGE_BLK
ROWS_PER_TILE = N_PAD // NUM_TILES
ROW_BLK = 128
R_BLK = 256

_SC_PARAMS = pltpu.CompilerParams(use_tc_tiling_on_sc=False)


def _sc_agg(d2):
    out_type = jax.ShapeDtypeStruct((NUM_CORES * N_PAD, d2), jnp.float32)
    scratch = [
        pltpu.VMEM_SHARED((N_PAD, d2), jnp.float32),
        pltpu.VMEM_SHARED((N_PAD, d2), jnp.float32),
        pltpu.VMEM((EDGE_BLK,), jnp.int32),
        pltpu.VMEM((EDGE_BLK,), jnp.int32),
        pltpu.VMEM((EDGE_BLK, d2), jnp.float32),
    ]

    def body(p_hbm, src_hbm, dst_hbm, acc_out, p_sp, acc_sp,
             src1, dst1, rows_v):
        c = lax.axis_index("c")
        s = lax.axis_index("s")
        rbase = s * ROWS_PER_TILE

        def zrow(i, _):
            def zcol(j, _):
                rows_v[i, pl.ds(j * 16, 16)] = jnp.zeros((16,), jnp.float32)
                return 0
            return lax.fori_loop(0, d2 // 16, zcol, 0)
        lax.fori_loop(0, EDGE_BLK, zrow, 0)
        for k in range(ROWS_PER_TILE // ROW_BLK):
            pltpu.sync_copy(rows_v,
                            acc_sp.at[pl.ds(rbase + k * ROW_BLK, ROW_BLK)])

        pltpu.sync_copy(p_hbm.at[pl.ds(c * N_PAD + rbase, ROWS_PER_TILE)],
                        p_sp.at[pl.ds(rbase, ROWS_PER_TILE)])
        plsc.subcore_barrier()

        def edge_blk(j, _):
            pltpu.sync_copy(src_hbm.at[s, pl.ds(j * EDGE_BLK, EDGE_BLK)],
                            src1)
            pltpu.sync_copy(dst_hbm.at[s, pl.ds(j * EDGE_BLK, EDGE_BLK)],
                            dst1)
            pltpu.sync_copy(p_sp.at[src1], rows_v)
            pltpu.sync_copy(rows_v, acc_sp.at[dst1], add=True)
            return 0
        lax.fori_loop(0, NBLK, edge_blk, 0)

        plsc.subcore_barrier()
        pltpu.sync_copy(acc_sp.at[pl.ds(rbase, ROWS_PER_TILE)],
                        acc_out.at[pl.ds(c * N_PAD + rbase, ROWS_PER_TILE)])

    mesh = plsc.VectorSubcoreMesh(core_axis_name="c", subcore_axis_name="s")
    return pl.kernel(body, out_type=out_type, mesh=mesh,
                     scratch_types=scratch, compiler_params=_SC_PARAMS)


def _dot(a, b):
    return jnp.dot(a, b, preferred_element_type=jnp.float32,
                   precision=lax.Precision.HIGHEST)


def _tc_first():
    def body(x_ref, ws_ref, wnp_ref, b_ref, oneh_ref, s_ref, p_ref):
        x = x_ref[...]
        s_ref[...] = _dot(x, ws_ref[...]) + b_ref[...]
        pp = _dot(x, wnp_ref[...]) + oneh_ref[...]
        p_ref[0] = pp[:, :80]
        p_ref[1] = pp[:, 80:]

    return pl.pallas_call(
        body,
        grid=(N_PAD // R_BLK,),
        in_specs=[
            pl.BlockSpec((R_BLK, 128), lambda i: (i, 0)),
            pl.BlockSpec((128, 128), lambda i: (0, 0)),
            pl.BlockSpec((128, 160), lambda i: (0, 0)),
            pl.BlockSpec((1, 128), lambda i: (0, 0)),
            pl.BlockSpec((1, 160), lambda i: (0, 0)),
        ],
        out_specs=[
            pl.BlockSpec((R_BLK, 128), lambda i: (i, 0)),
            pl.BlockSpec((2, R_BLK, 80), lambda i: (0, i, 0)),
        ],
        out_shape=[
            jax.ShapeDtypeStruct((N_PAD, 128), jnp.float32),
            jax.ShapeDtypeStruct((2, N_PAD, 80), jnp.float32),
        ],
    )


def _combine(s_blk, acc_blk, deg_blk, f):
    invdeg = 1.0 / jnp.maximum(deg_blk, 1.0)
    neigh = jnp.concatenate([acc_blk[0, :, :f], acc_blk[1, :, :f]], axis=-1)
    return s_blk + neigh * invdeg


def _tc_mid(d2p, f, d_h):
    d2 = d_h // 2

    def body(s_in_ref, acc_ref, deg_ref, ws_ref, wn_ref, b_ref,
             s_ref, p_ref):
        h = _combine(s_in_ref[...], acc_ref[...], deg_ref[...], f)
        h = jnp.maximum(h, 0.0)
        s_ref[...] = _dot(h, ws_ref[...]) + b_ref[...]
        p = _dot(h, wn_ref[...])
        p_ref[0] = p[:, :d2]
        p_ref[1] = p[:, d2:]

    return pl.pallas_call(
        body,
        grid=(N_PAD // R_BLK,),
        in_specs=[
            pl.BlockSpec((R_BLK, 128), lambda i: (i, 0)),
            pl.BlockSpec((2, R_BLK, d2p), lambda i: (0, i, 0)),
            pl.BlockSpec((R_BLK, 1), lambda i: (i, 0)),
            pl.BlockSpec((128, d_h), lambda i: (0, 0)),
            pl.BlockSpec((128, d_h), lambda i: (0, 0)),
            pl.BlockSpec((1, d_h), lambda i: (0, 0)),
        ],
        out_specs=[
            pl.BlockSpec((R_BLK, d_h), lambda i: (i, 0)),
            pl.BlockSpec((2, R_BLK, d2), lambda i: (0, i, 0)),
        ],
        out_shape=[
            jax.ShapeDtypeStruct((N_PAD, d_h), jnp.float32),
            jax.ShapeDtypeStruct((2, N_PAD, d2), jnp.float32),
        ],
    )


def _tc_final():
    def body(s_in_ref, acc_ref, deg_ref, out_ref):
        out_ref[...] = _combine(s_in_ref[...], acc_ref[...], deg_ref[...], 32)

    return pl.pallas_call(
        body,
        grid=(N_PAD // R_BLK,),
        in_specs=[
            pl.BlockSpec((R_BLK, 64), lambda i: (i, 0)),
            pl.BlockSpec((2, R_BLK, 32), lambda i: (0, i, 0)),
            pl.BlockSpec((R_BLK, 1), lambda i: (i, 0)),
        ],
        out_specs=pl.BlockSpec((R_BLK, 64), lambda i: (i, 0)),
        out_shape=jax.ShapeDtypeStruct((N_PAD, 64), jnp.float32),
    )


@jax.jit
def _run(x, edge_index, Ws0, Wn0, b0, Ws1, Wn1, b1, Ws2, Wn2, b2):
    x_pad = jnp.zeros((N_PAD, 128), x.dtype).at[:N_NODES].set(x)
    n_extra = E_PAD - N_EDGES
    fill = (N_NODES + jnp.arange(n_extra, dtype=jnp.int32)
            % (N_PAD - N_NODES)).astype(jnp.int32)
    src = jnp.concatenate([edge_index[0], fill]).reshape(NUM_TILES, -1)
    dst = jnp.concatenate([edge_index[1], fill]).reshape(NUM_TILES, -1)

    Wn0p = jnp.zeros((128, 160), jnp.float32)
    Wn0p = Wn0p.at[:, 0:64].set(Wn0[:, 0:64]).at[:, 80:144].set(Wn0[:, 64:])
    oneh = jnp.zeros((1, 160), jnp.float32).at[0, 64].set(1.0)
    oneh = oneh.at[0, 144].set(1.0)

    s0, p0 = _tc_first()(x_pad, Ws0, Wn0p, b0.reshape(1, -1), oneh)
    acc0 = _sc_agg(80)(p0.reshape(2 * N_PAD, 80), src, dst)
    acc0 = acc0.reshape(2, N_PAD, 80)
    deg = acc0[0, :, 64:65]
    s1, p1 = _tc_mid(80, 64, 128)(s0, acc0, deg, Ws1, Wn1, b1.reshape(1, -1))
    acc1 = _sc_agg(64)(p1.reshape(2 * N_PAD, 64), src, dst)
    s2, p2 = _tc_mid(64, 64, 64)(s1, acc1.reshape(2, N_PAD, 64), deg,
                                 Ws2, Wn2, b2.reshape(1, -1))
    acc2 = _sc_agg(32)(p2.reshape(2 * N_PAD, 32), src, dst)
    out = _tc_final()(s2, acc2.reshape(2, N_PAD, 32), deg)
    return out[:N_NODES]


def kernel(x, edge_index, Ws0, Wn0, b0, Ws1, Wn1, b1, Ws2, Wn2, b2):
    return _run(x, edge_index, Ws0, Wn0, b0, Ws1, Wn1, b1, Ws2, Wn2, b2)


# speedup vs baseline: 7.4940x; 1.5851x over previous
import jax
import jax.numpy as jnp
from jax import lax
from jax.experimental import pallas as pl
from jax.experimental.pallas import tpu as pltpu
from jax.experimental.pallas import tpu_sc as plsc

N_NODES = 10000
N_PAD = 10240
N_EDGES = 320000
NUM_TILES = 16
NUM_CORES = 2
EDGE_BLK = 128
NBLK = 160
CHUNK = 8
E_PAD = NUM_TILES * NBLK * EDGE_BLK
ROWS_PER_TILE = N_PAD // NUM_TILES
ROW_BLK = 128
DEG_W = 16
R_BLK = 256

_SC_PARAMS = pltpu.CompilerParams(use_tc_tiling_on_sc=False)


def _sc_agg(d2, with_deg):
    out_type = [jax.ShapeDtypeStruct((NUM_CORES * N_PAD, d2), jnp.float32)]
    scratch = [
        pltpu.VMEM_SHARED((N_PAD, d2), jnp.float32),
        pltpu.VMEM_SHARED((N_PAD, d2), jnp.float32),
        pltpu.VMEM((CHUNK, EDGE_BLK), jnp.int32),
        pltpu.VMEM((CHUNK, EDGE_BLK), jnp.int32),
        pltpu.VMEM((EDGE_BLK, d2), jnp.float32),
        pltpu.VMEM((EDGE_BLK, d2), jnp.float32),
        pltpu.SemaphoreType.DMA,
        pltpu.SemaphoreType.DMA,
        pltpu.SemaphoreType.DMA,
    ]
    if with_deg:
        out_type.append(
            jax.ShapeDtypeStruct((NUM_CORES * N_PAD, DEG_W), jnp.float32))
        scratch.append(pltpu.VMEM_SHARED((N_PAD, DEG_W), jnp.float32))
        scratch.append(pltpu.VMEM((EDGE_BLK, DEG_W), jnp.float32))
        scratch.append(pltpu.SemaphoreType.DMA)

    def body(p_hbm, src_hbm, dst_hbm, acc_out, *rest):
        if with_deg:
            (degp_out, p_sp, acc_sp, src_v, dst_v, rows0, rows1,
             ssem0, ssem1, gsem, deg_sp, ones_v, dsem) = rest
        else:
            (p_sp, acc_sp, src_v, dst_v, rows0, rows1,
             ssem0, ssem1, gsem) = rest
        c = lax.axis_index("c")
        s = lax.axis_index("s")
        rbase = s * ROWS_PER_TILE
        rows = (rows0, rows1)
        ssem = (ssem0, ssem1)

        def fillzero(buf, w):
            def zr(i, _):
                def zc(j, _):
                    buf[i, pl.ds(j * 16, 16)] = jnp.zeros((16,), jnp.float32)
                    return 0
                return lax.fori_loop(0, w // 16, zc, 0)
            lax.fori_loop(0, EDGE_BLK, zr, 0)

        fillzero(rows0, d2)
        for k in range(ROWS_PER_TILE // ROW_BLK):
            pltpu.sync_copy(rows0,
                            acc_sp.at[pl.ds(rbase + k * ROW_BLK, ROW_BLK)])
        if with_deg:
            fillzero(ones_v, DEG_W)
            for k in range(ROWS_PER_TILE // ROW_BLK):
                pltpu.sync_copy(
                    ones_v, deg_sp.at[pl.ds(rbase + k * ROW_BLK, ROW_BLK)])
            one0 = jnp.where(lax.iota(jnp.int32, 16) == 0,
                             jnp.float32(1.0), jnp.float32(0.0))

            def srow(i, _):
                ones_v[i, pl.ds(0, 16)] = one0
                return 0
            lax.fori_loop(0, EDGE_BLK, srow, 0)

        pltpu.sync_copy(p_hbm.at[pl.ds(c * N_PAD + rbase, ROWS_PER_TILE)],
                        p_sp.at[pl.ds(rbase, ROWS_PER_TILE)])
        plsc.subcore_barrier()

        for p2 in range(2):
            pltpu.async_copy(p_sp.at[pl.ds(rbase, EDGE_BLK)], rows[p2],
                             ssem[p2])

        def chunk_loop(ci, _):
            pltpu.sync_copy(src_hbm.at[s, pl.ds(ci * CHUNK, CHUNK)], src_v)
            pltpu.sync_copy(dst_hbm.at[s, pl.ds(ci * CHUNK, CHUNK)], dst_v)
            for jj in range(CHUNK):
                p2 = jj % 2
                sidx = src_v.at[jj]
                didx = dst_v.at[jj]
                pltpu.make_async_copy(rows[p2], acc_sp.at[didx],
                                      ssem[p2]).wait()
                pltpu.async_copy(p_sp.at[sidx], rows[p2], gsem).wait()
                pltpu.async_copy(rows[p2], acc_sp.at[didx], ssem[p2],
                                 add=True)
                if with_deg:
                    pltpu.async_copy(ones_v, deg_sp.at[didx], dsem, add=True)
            return 0
        lax.fori_loop(0, NBLK // CHUNK, chunk_loop, 0)

        for p2 in range(2):
            pltpu.make_async_copy(rows[p2], acc_sp.at[dst_v.at[0]],
                                  ssem[p2]).wait()
        if with_deg:
            def drain(i, _):
                pltpu.make_async_copy(ones_v, deg_sp.at[dst_v.at[0]],
                                      dsem).wait()
                return 0
            lax.fori_loop(0, NBLK, drain, 0)

        plsc.subcore_barrier()
        pltpu.sync_copy(acc_sp.at[pl.ds(rbase, ROWS_PER_TILE)],
                        acc_out.at[pl.ds(c * N_PAD + rbase, ROWS_PER_TILE)])
        if with_deg:
            pltpu.sync_copy(
                deg_sp.at[pl.ds(rbase, ROWS_PER_TILE)],
                degp_out.at[pl.ds(c * N_PAD + rbase, ROWS_PER_TILE)])

    mesh = plsc.VectorSubcoreMesh(core_axis_name="c", subcore_axis_name="s")
    return pl.kernel(body, out_type=tuple(out_type), mesh=mesh,
                     scratch_types=scratch, compiler_params=_SC_PARAMS)


def _dot(a, b):
    return jnp.dot(a, b, preferred_element_type=jnp.float32,
                   precision=lax.Precision.HIGHEST)


def _tc_first():
    def body(x_ref, ws_ref, wn_ref, b_ref, s_ref, p_ref):
        x = x_ref[...]
        s_ref[...] = _dot(x, ws_ref[...]) + b_ref[...]
        p = _dot(x, wn_ref[...])
        p_ref[0] = p[:, :64]
        p_ref[1] = p[:, 64:]

    return pl.pallas_call(
        body,
        grid=(N_PAD // R_BLK,),
        in_specs=[
            pl.BlockSpec((R_BLK, 128), lambda i: (i, 0)),
            pl.BlockSpec((128, 128), lambda i: (0, 0)),
            pl.BlockSpec((128, 128), lambda i: (0, 0)),
            pl.BlockSpec((1, 128), lambda i: (0, 0)),
        ],
        out_specs=[
            pl.BlockSpec((R_BLK, 128), lambda i: (i, 0)),
            pl.BlockSpec((2, R_BLK, 64), lambda i: (0, i, 0)),
        ],
        out_shape=[
            jax.ShapeDtypeStruct((N_PAD, 128), jnp.float32),
            jax.ShapeDtypeStruct((2, N_PAD, 64), jnp.float32),
        ],
    )


def _combine(s_blk, acc_blk, deg_blk, f):
    invdeg = 1.0 / jnp.maximum(deg_blk[:, 0:1], 1.0)
    neigh = jnp.concatenate([acc_blk[0, :, :f], acc_blk[1, :, :f]], axis=-1)
    return s_blk + neigh * invdeg


def _tc_mid(d2p, f, d_h):
    d2 = d_h // 2

    def body(s_in_ref, acc_ref, deg_ref, ws_ref, wn_ref, b_ref,
             s_ref, p_ref):
        h = _combine(s_in_ref[...], acc_ref[...], deg_ref[...], f)
        h = jnp.maximum(h, 0.0)
        s_ref[...] = _dot(h, ws_ref[...]) + b_ref[...]
        p = _dot(h, wn_ref[...])
        p_ref[0] = p[:, :d2]
        p_ref[1] = p[:, d2:]

    return pl.pallas_call(
        body,
        grid=(N_PAD // R_BLK,),
        in_specs=[
            pl.BlockSpec((R_BLK, 128), lambda i: (i, 0)),
            pl.BlockSpec((2, R_BLK, d2p), lambda i: (0, i, 0)),
            pl.BlockSpec((R_BLK, DEG_W), lambda i: (i, 0)),
            pl.BlockSpec((128, d_h), lambda i: (0, 0)),
            pl.BlockSpec((128, d_h), lambda i: (0, 0)),
            pl.BlockSpec((1, d_h), lambda i: (0, 0)),
        ],
        out_specs=[
            pl.BlockSpec((R_BLK, d_h), lambda i: (i, 0)),
            pl.BlockSpec((2, R_BLK, d2), lambda i: (0, i, 0)),
        ],
        out_shape=[
            jax.ShapeDtypeStruct((N_PAD, d_h), jnp.float32),
            jax.ShapeDtypeStruct((2, N_PAD, d2), jnp.float32),
        ],
    )


def _tc_final():
    def body(s_in_ref, acc_ref, deg_ref, out_ref):
        out_ref[...] = _combine(s_in_ref[...], acc_ref[...], deg_ref[...], 32)

    return pl.pallas_call(
        body,
        grid=(N_PAD // R_BLK,),
        in_specs=[
            pl.BlockSpec((R_BLK, 64), lambda i: (i, 0)),
            pl.BlockSpec((2, R_BLK, 32), lambda i: (0, i, 0)),
            pl.BlockSpec((R_BLK, DEG_W), lambda i: (i, 0)),
        ],
        out_specs=pl.BlockSpec((R_BLK, 64), lambda i: (i, 0)),
        out_shape=jax.ShapeDtypeStruct((N_PAD, 64), jnp.float32),
    )


@jax.jit
def _run(x, edge_index, Ws0, Wn0, b0, Ws1, Wn1, b1, Ws2, Wn2, b2):
    x_pad = jnp.zeros((N_PAD, 128), x.dtype).at[:N_NODES].set(x)
    n_extra = E_PAD - N_EDGES
    fill = (N_NODES + jnp.arange(n_extra, dtype=jnp.int32)
            % (N_PAD - N_NODES)).astype(jnp.int32)
    src = jnp.concatenate([edge_index[0], fill]).reshape(
        NUM_TILES, NBLK, EDGE_BLK)
    dst = jnp.concatenate([edge_index[1], fill]).reshape(
        NUM_TILES, NBLK, EDGE_BLK)

    s0, p0 = _tc_first()(x_pad, Ws0, Wn0, b0.reshape(1, -1))
    acc0, degp = _sc_agg(64, True)(p0.reshape(2 * N_PAD, 64), src, dst)
    deg0 = degp[:N_PAD]
    s1, p1 = _tc_mid(64, 64, 128)(s0, acc0.reshape(2, N_PAD, 64), deg0,
                                  Ws1, Wn1, b1.reshape(1, -1))
    (acc1,) = _sc_agg(64, False)(p1.reshape(2 * N_PAD, 64), src, dst)
    s2, p2 = _tc_mid(64, 64, 64)(s1, acc1.reshape(2, N_PAD, 64), deg0,
                                 Ws2, Wn2, b2.reshape(1, -1))
    (acc2,) = _sc_agg(32, False)(p2.reshape(2 * N_PAD, 32), src, dst)
    out = _tc_final()(s2, acc2.reshape(2, N_PAD, 32), deg0)
    return out[:N_NODES]


def kernel(x, edge_index, Ws0, Wn0, b0, Ws1, Wn1, b1, Ws2, Wn2, b2):
    return _run(x, edge_index, Ws0, Wn0, b0, Ws1, Wn1, b1, Ws2, Wn2, b2)


# speedup vs baseline: 8.7083x; 1.1620x over previous
import jax
import jax.numpy as jnp
from jax import lax
from jax.experimental import pallas as pl
from jax.experimental.pallas import tpu as pltpu
from jax.experimental.pallas import tpu_sc as plsc

N_NODES = 10000
N_PAD = 10240
N_EDGES = 320000
NUM_TILES = 16
NUM_CORES = 2
EDGE_BLK = 128
NBLK = 160
CHUNK = 8
NCHUNK = NBLK // CHUNK
NBLK_ARR = NBLK + CHUNK
E_PAD = NUM_TILES * NBLK * EDGE_BLK
ROWS_PER_TILE = N_PAD // NUM_TILES
ROW_BLK = 128
R_BLK = 256

_SC_PARAMS = pltpu.CompilerParams(use_tc_tiling_on_sc=False)


def _sc_agg(d2):
    out_type = jax.ShapeDtypeStruct((NUM_CORES * N_PAD, d2), jnp.float32)
    scratch = [
        pltpu.VMEM_SHARED((N_PAD, d2), jnp.float32),
        pltpu.VMEM_SHARED((N_PAD, d2), jnp.float32),
        pltpu.VMEM((CHUNK, EDGE_BLK), jnp.int32),
        pltpu.VMEM((CHUNK, EDGE_BLK), jnp.int32),
        pltpu.VMEM((CHUNK, EDGE_BLK), jnp.int32),
        pltpu.VMEM((CHUNK, EDGE_BLK), jnp.int32),
        pltpu.VMEM((EDGE_BLK, d2), jnp.float32),
        pltpu.VMEM((EDGE_BLK, d2), jnp.float32),
        pltpu.SemaphoreType.DMA,
        pltpu.SemaphoreType.DMA,
        pltpu.SemaphoreType.DMA,
        pltpu.SemaphoreType.DMA,
        pltpu.SemaphoreType.DMA,
    ]

    def body(p_hbm, src_hbm, dst_hbm, acc_out, p_sp, acc_sp,
             src0, src1, dst0, dst1, rows0, rows1,
             ssem0, ssem1, gsem, isem0, isem1):
        c = lax.axis_index("c")
        s = lax.axis_index("s")
        rbase = s * ROWS_PER_TILE
        rows = (rows0, rows1)
        ssem = (ssem0, ssem1)
        srcb = (src0, src1)
        dstb = (dst0, dst1)
        isem = (isem0, isem1)

        def zr(i, _):
            def zc(j, _):
                rows0[i, pl.ds(j * 16, 16)] = jnp.zeros((16,), jnp.float32)
                return 0
            return lax.fori_loop(0, d2 // 16, zc, 0)
        lax.fori_loop(0, EDGE_BLK, zr, 0)
        for k in range(ROWS_PER_TILE // ROW_BLK):
            pltpu.sync_copy(rows0,
                            acc_sp.at[pl.ds(rbase + k * ROW_BLK, ROW_BLK)])

        pltpu.sync_copy(p_hbm.at[pl.ds(c * N_PAD + rbase, ROWS_PER_TILE)],
                        p_sp.at[pl.ds(rbase, ROWS_PER_TILE)])
        plsc.subcore_barrier()

        for p2 in range(2):
            pltpu.async_copy(p_sp.at[pl.ds(rbase, EDGE_BLK)], rows[p2],
                             ssem[p2])
        pltpu.async_copy(src_hbm.at[s, pl.ds(0, CHUNK)], src0, isem0)
        pltpu.async_copy(dst_hbm.at[s, pl.ds(0, CHUNK)], dst0, isem0)

        def idx_wait(b):
            pltpu.make_async_copy(src_hbm.at[s, pl.ds(0, CHUNK)],
                                  srcb[b], isem[b]).wait()
            pltpu.make_async_copy(dst_hbm.at[s, pl.ds(0, CHUNK)],
                                  dstb[b], isem[b]).wait()

        def chunk_pair(k, _):
            for h in range(2):
                b = h
                nb = 1 - h
                ci = 2 * k + h
                idx_wait(b)
                pltpu.async_copy(
                    src_hbm.at[s, pl.ds((ci + 1) * CHUNK, CHUNK)],
                    srcb[nb], isem[nb])
                pltpu.async_copy(
                    dst_hbm.at[s, pl.ds((ci + 1) * CHUNK, CHUNK)],
                    dstb[nb], isem[nb])
                for jj in range(CHUNK):
                    p2 = jj % 2
                    sidx = srcb[b].at[jj]
                    didx = dstb[b].at[jj]
                    pltpu.make_async_copy(rows[p2], acc_sp.at[didx],
                                          ssem[p2]).wait()
                    pltpu.async_copy(p_sp.at[sidx], rows[p2], gsem).wait()
                    pltpu.async_copy(rows[p2], acc_sp.at[didx], ssem[p2],
                                     add=True)
            return 0
        lax.fori_loop(0, NCHUNK // 2, chunk_pair, 0)

        for p2 in range(2):
            pltpu.make_async_copy(rows[p2], acc_sp.at[dst0.at[0]],
                                  ssem[p2]).wait()
        idx_wait(0)

        plsc.subcore_barrier()
        pltpu.sync_copy(acc_sp.at[pl.ds(rbase, ROWS_PER_TILE)],
                        acc_out.at[pl.ds(c * N_PAD + rbase, ROWS_PER_TILE)])

    mesh = plsc.VectorSubcoreMesh(core_axis_name="c", subcore_axis_name="s")
    return pl.kernel(body, out_type=out_type, mesh=mesh,
                     scratch_types=scratch, compiler_params=_SC_PARAMS)


def _dot(a, b):
    return jnp.dot(a, b, preferred_element_type=jnp.float32,
                   precision=lax.Precision.HIGHEST)


def _tc_first():
    def body(x_ref, ws_ref, wnp_ref, b_ref, oneh_ref, s_ref, p_ref):
        x = x_ref[...]
        s_ref[...] = _dot(x, ws_ref[...]) + b_ref[...]
        pp = _dot(x, wnp_ref[...]) + oneh_ref[...]
        p_ref[0] = pp[:, :80]
        p_ref[1] = pp[:, 80:]

    return pl.pallas_call(
        body,
        grid=(N_PAD // R_BLK,),
        in_specs=[
            pl.BlockSpec((R_BLK, 128), lambda i: (i, 0)),
            pl.BlockSpec((128, 128), lambda i: (0, 0)),
            pl.BlockSpec((128, 160), lambda i: (0, 0)),
            pl.BlockSpec((1, 128), lambda i: (0, 0)),
            pl.BlockSpec((1, 160), lambda i: (0, 0)),
        ],
        out_specs=[
            pl.BlockSpec((R_BLK, 128), lambda i: (i, 0)),
            pl.BlockSpec((2, R_BLK, 80), lambda i: (0, i, 0)),
        ],
        out_shape=[
            jax.ShapeDtypeStruct((N_PAD, 128), jnp.float32),
            jax.ShapeDtypeStruct((2, N_PAD, 80), jnp.float32),
        ],
    )


def _tc_mid1():
    def body(s_in_ref, acc_ref, ws_ref, wn_ref, b_ref, s_ref, p_ref):
        acc = acc_ref[...]
        invdeg = 1.0 / jnp.maximum(acc[0, :, 64:65], 1.0)
        neigh = jnp.concatenate([acc[0, :, :64], acc[1, :, :64]], axis=-1)
        h = jnp.maximum(s_in_ref[...] + neigh * invdeg, 0.0)
        s_ref[...] = _dot(h, ws_ref[...]) + b_ref[...]
        p = _dot(h, wn_ref[...])
        p_ref[0] = p[:, :64]
        p_ref[1] = p[:, 64:]

    return pl.pallas_call(
        body,
        grid=(N_PAD // R_BLK,),
        in_specs=[
            pl.BlockSpec((R_BLK, 128), lambda i: (i, 0)),
            pl.BlockSpec((2, R_BLK, 80), lambda i: (0, i, 0)),
            pl.BlockSpec((128, 128), lambda i: (0, 0)),
            pl.BlockSpec((128, 128), lambda i: (0, 0)),
            pl.BlockSpec((1, 128), lambda i: (0, 0)),
        ],
        out_specs=[
            pl.BlockSpec((R_BLK, 128), lambda i: (i, 0)),
            pl.BlockSpec((2, R_BLK, 64), lambda i: (0, i, 0)),
        ],
        out_shape=[
            jax.ShapeDtypeStruct((N_PAD, 128), jnp.float32),
            jax.ShapeDtypeStruct((2, N_PAD, 64), jnp.float32),
        ],
    )


def _tc_mid2():
    def body(s_in_ref, acc_ref, deg_ref, ws_ref, wn_ref, b_ref,
             s_ref, p_ref):
        acc = acc_ref[...]
        invdeg = 1.0 / jnp.maximum(deg_ref[...], 1.0)
        neigh = jnp.concatenate([acc[0], acc[1]], axis=-1)
        h = jnp.maximum(s_in_ref[...] + neigh * invdeg, 0.0)
        s_ref[...] = _dot(h, ws_ref[...]) + b_ref[...]
        p = _dot(h, wn_ref[...])
        p_ref[0] = p[:, :32]
        p_ref[1] = p[:, 32:]

    return pl.pallas_call(
        body,
        grid=(N_PAD // R_BLK,),
        in_specs=[
            pl.BlockSpec((R_BLK, 128), lambda i: (i, 0)),
            pl.BlockSpec((2, R_BLK, 64), lambda i: (0, i, 0)),
            pl.BlockSpec((R_BLK, 1), lambda i: (i, 0)),
            pl.BlockSpec((128, 64), lambda i: (0, 0)),
            pl.BlockSpec((128, 64), lambda i: (0, 0)),
            pl.BlockSpec((1, 64), lambda i: (0, 0)),
        ],
        out_specs=[
            pl.BlockSpec((R_BLK, 64), lambda i: (i, 0)),
            pl.BlockSpec((2, R_BLK, 32), lambda i: (0, i, 0)),
        ],
        out_shape=[
            jax.ShapeDtypeStruct((N_PAD, 64), jnp.float32),
            jax.ShapeDtypeStruct((2, N_PAD, 32), jnp.float32),
        ],
    )


def _tc_final():
    def body(s_in_ref, acc_ref, deg_ref, out_ref):
        acc = acc_ref[...]
        invdeg = 1.0 / jnp.maximum(deg_ref[...], 1.0)
        neigh = jnp.concatenate([acc[0], acc[1]], axis=-1)
        out_ref[...] = s_in_ref[...] + neigh * invdeg

    return pl.pallas_call(
        body,
        grid=(N_PAD // R_BLK,),
        in_specs=[
            pl.BlockSpec((R_BLK, 64), lambda i: (i, 0)),
            pl.BlockSpec((2, R_BLK, 32), lambda i: (0, i, 0)),
            pl.BlockSpec((R_BLK, 1), lambda i: (i, 0)),
        ],
        out_specs=pl.BlockSpec((R_BLK, 64), lambda i: (i, 0)),
        out_shape=jax.ShapeDtypeStruct((N_PAD, 64), jnp.float32),
    )


@jax.jit
def _run(x, edge_index, Ws0, Wn0, b0, Ws1, Wn1, b1, Ws2, Wn2, b2):
    x_pad = jnp.zeros((N_PAD, 128), x.dtype).at[:N_NODES].set(x)
    n_extra = E_PAD - N_EDGES
    fill = (N_NODES + jnp.arange(n_extra, dtype=jnp.int32)
            % (N_PAD - N_NODES)).astype(jnp.int32)

    def shape_idx(a):
        a = jnp.concatenate([a, fill]).reshape(NUM_TILES, NBLK, EDGE_BLK)
        pad = jnp.zeros((NUM_TILES, CHUNK, EDGE_BLK), jnp.int32)
        return jnp.concatenate([a, pad], axis=1)

    src = shape_idx(edge_index[0])
    dst = shape_idx(edge_index[1])

    Wn0p = jnp.zeros((128, 160), jnp.float32)
    Wn0p = Wn0p.at[:, 0:64].set(Wn0[:, 0:64]).at[:, 80:144].set(Wn0[:, 64:])
    oneh = jnp.zeros((1, 160), jnp.float32).at[0, 64].set(1.0)
    oneh = oneh.at[0, 144].set(1.0)

    s0, p0 = _tc_first()(x_pad, Ws0, Wn0p, b0.reshape(1, -1), oneh)
    acc0 = _sc_agg(80)(p0.reshape(2 * N_PAD, 80), src, dst)
    acc0 = acc0.reshape(2, N_PAD, 80)
    deg0 = acc0[0, :, 64:65]
    s1, p1 = _tc_mid1()(s0, acc0, Ws1, Wn1, b1.reshape(1, -1))
    acc1 = _sc_agg(64)(p1.reshape(2 * N_PAD, 64), src, dst)
    s2, p2 = _tc_mid2()(s1, acc1.reshape(2, N_PAD, 64), deg0,
                        Ws2, Wn2, b2.reshape(1, -1))
    acc2 = _sc_agg(32)(p2.reshape(2 * N_PAD, 32), src, dst)
    out = _tc_final()(s2, acc2.reshape(2, N_PAD, 32), deg0)
    return out[:N_NODES]


def kernel(x, edge_index, Ws0, Wn0, b0, Ws1, Wn1, b1, Ws2, Wn2, b2):
    return _run(x, edge_index, Ws0, Wn0, b0, Ws1, Wn1, b1, Ws2, Wn2, b2)
